# Initial kernel scaffold; baseline (speedup 1.0000x reference)
import jax
import jax.numpy as jnp
from jax.experimental import pallas as pl


def kernel(x, edge_index, W1, b1, W2, b2):
    raise NotImplementedError("write your pallas kernel here")


# speedup vs baseline: 42.9565x; 42.9565x over previous
import functools

import jax
import jax.numpy as jnp
from jax import lax
from jax.experimental import pallas as pl
from jax.experimental.pallas import tpu as pltpu
from jax.experimental.pallas import tpu_sc as plsc

_F32 = jnp.float32
_NT = 32
_NSUB = 16


def _sc_mesh():
    return plsc.VectorSubcoreMesh(core_axis_name="c", subcore_axis_name="s")


@functools.lru_cache(maxsize=None)
def _build_deg_kernel(E, NP):
    EPT = E // _NT

    def body(dst_hbm, out_hbm, dstb, acc):
        cid = lax.axis_index("c")
        sid = lax.axis_index("s")
        t = cid * _NSUB + sid
        zero = jnp.zeros((16,), _F32)

        def zb(i, c):
            acc[pl.ds(i * 16, 16)] = zero
            return c

        lax.fori_loop(0, NP // 16, zb, 0)
        pltpu.sync_copy(dst_hbm.at[pl.ds(t * EPT, EPT)], dstb)
        ones = jnp.ones((16,), _F32)

        def eb(i, c):
            idx = dstb[pl.ds(i * 16, 16)]
            plsc.addupdate_scatter(acc, [idx], ones)
            return c

        lax.fori_loop(0, EPT // 16, eb, 0)
        pltpu.sync_copy(acc, out_hbm.at[t])

    return pl.kernel(
        body,
        out_type=jax.ShapeDtypeStruct((_NT, NP), _F32),
        mesh=_sc_mesh(),
        compiler_params=pltpu.CompilerParams(needs_layout_passes=False),
        scratch_types=[
            pltpu.VMEM((EPT,), jnp.int32),
            pltpu.VMEM((NP,), _F32),
        ],
    )


def _tc_prep(p_t, x_pad, W1):
    NP = x_pad.shape[0]
    DH = W1.shape[1]

    def body(p_ref, x_ref, w1_ref, g1_ref, dis_ref):
        deg = jnp.sum(p_ref[...], axis=1, keepdims=True) + 1.0
        dis = lax.rsqrt(deg)
        h1 = jnp.dot(x_ref[...], w1_ref[...], preferred_element_type=_F32)
        g1_ref[...] = h1 * dis
        dis_ref[...] = dis

    return pl.pallas_call(
        body,
        out_shape=[
            jax.ShapeDtypeStruct((NP, DH), _F32),
            jax.ShapeDtypeStruct((NP, 1), _F32),
        ],
    )(p_t, x_pad, W1)


@functools.lru_cache(maxsize=None)
def _build_prop1_kernel(E, NP, DH, CH):
    NIT = (E // _NT) // CH
    NPT = NP // _NSUB

    def body(src_hbm, dst_hbm, g1_hbm, out_hbm,
             srcb, dstb, rows0, rows1, zbuf, sacc, sg0, sg1, ss0, ss1):
        cid = lax.axis_index("c")
        sid = lax.axis_index("s")
        t = cid * _NSUB + sid
        pltpu.sync_copy(src_hbm.at[pl.ds(t * NIT, NIT)], srcb)
        pltpu.sync_copy(dst_hbm.at[pl.ds(t * NIT, NIT)], dstb)
        zero = jnp.zeros((16,), _F32)

        def zb(i, c):
            def zc(j, c2):
                zbuf[i, pl.ds(j * 16, 16)] = zero
                return c2
            lax.fori_loop(0, DH // 16, zc, 0)
            return c

        lax.fori_loop(0, 128, zb, 0)
        for r in range(NPT // 128):
            pltpu.sync_copy(zbuf, sacc.at[pl.ds(sid * NPT + r * 128, 128)])
        plsc.subcore_barrier()

        def start_g(i, rbuf, sem):
            pltpu.async_copy(g1_hbm.at[srcb.at[i]], rbuf, sem)

        def wait_g(rbuf, sem):
            pltpu.make_async_copy(g1_hbm.at[srcb.at[0]], rbuf, sem).wait()

        def start_s(i, rbuf, sem):
            pltpu.async_copy(rbuf, sacc.at[dstb.at[i]], sem, add=True)

        def wait_s(rbuf, sem):
            pltpu.make_async_copy(rbuf, sacc.at[dstb.at[0]], sem).wait()

        start_g(0, rows0, sg0)
        start_g(1, rows1, sg1)

        def eb(j, c):
            i0 = 2 * j
            wait_g(rows0, sg0)
            start_s(i0, rows0, ss0)
            wait_g(rows1, sg1)
            start_s(i0 + 1, rows1, ss1)

            @pl.when(j < NIT // 2 - 1)
            def _():
                wait_s(rows0, ss0)
                start_g(i0 + 2, rows0, sg0)
                wait_s(rows1, ss1)
                start_g(i0 + 3, rows1, sg1)

            return c

        lax.fori_loop(0, NIT // 2, eb, 0)
        wait_s(rows0, ss0)
        wait_s(rows1, ss1)
        plsc.subcore_barrier()
        pltpu.sync_copy(sacc.at[pl.ds(sid * NPT, NPT)],
                        out_hbm.at[cid, pl.ds(sid * NPT, NPT)])

    return pl.kernel(
        body,
        out_type=jax.ShapeDtypeStruct((2, NP, DH), _F32),
        mesh=_sc_mesh(),
        compiler_params=pltpu.CompilerParams(
            needs_layout_passes=False, use_tc_tiling_on_sc=False),
        scratch_types=[
            pltpu.VMEM((NIT, CH), jnp.int32),
            pltpu.VMEM((NIT, CH), jnp.int32),
            pltpu.VMEM((CH, DH), _F32),
            pltpu.VMEM((CH, DH), _F32),
            pltpu.VMEM((128, DH), _F32),
            pltpu.VMEM_SHARED((NP, DH), _F32),
            pltpu.SemaphoreType.DMA,
            pltpu.SemaphoreType.DMA,
            pltpu.SemaphoreType.DMA,
            pltpu.SemaphoreType.DMA,
        ],
    )


def _tc_mid(accp, g1, dis2d, b1, W2):
    NP, DH = g1.shape

    def body(a_ref, g1_ref, dis_ref, b1_ref, w2_ref, g2_ref):
        acc = a_ref[0] + a_ref[1] + g1_ref[...]
        dis = dis_ref[...]
        out1 = acc * dis + b1_ref[...]
        h = jnp.maximum(out1, 0.0)
        h2 = jnp.dot(h, w2_ref[...], preferred_element_type=_F32)
        g2_ref[...] = dis * h2

    return pl.pallas_call(
        body,
        out_shape=jax.ShapeDtypeStruct((NP, 1), _F32),
    )(accp, g1, dis2d, b1.reshape(1, DH), W2)


@functools.lru_cache(maxsize=None)
def _build_prop2_kernel(E, NP):
    EPT = E // _NSUB
    NPT = NP // _NSUB

    def body(g2_hbm, dis_hbm, src_hbm, dst_hbm, b2_hbm, out_hbm,
             g2b, srcb, dstb, acc2, disb, outb, b2b, redb, slots):
        cid = lax.axis_index("c")
        sid = lax.axis_index("s")

        @pl.when(cid == 0)
        def _():
            pltpu.sync_copy(g2_hbm, g2b)
            zero = jnp.zeros((16,), _F32)

            def zb(i, c):
                acc2[pl.ds(i * 16, 16)] = zero
                return c

            lax.fori_loop(0, NP // 16, zb, 0)
            pltpu.sync_copy(src_hbm.at[pl.ds(sid * EPT, EPT)], srcb)
            pltpu.sync_copy(dst_hbm.at[pl.ds(sid * EPT, EPT)], dstb)

            def eb(i, c):
                s = srcb[pl.ds(i * 16, 16)]
                d = dstb[pl.ds(i * 16, 16)]
                v = plsc.load_gather(g2b, [s])
                plsc.addupdate_scatter(acc2, [d], v)
                return c

            lax.fori_loop(0, EPT // 16, eb, 0)
            pltpu.sync_copy(acc2, slots.at[sid])
            plsc.subcore_barrier()
            pltpu.sync_copy(dis_hbm.at[pl.ds(sid * NPT, NPT)], disb)
            pltpu.sync_copy(b2_hbm, b2b)
            for j in range(_NSUB):
                pltpu.sync_copy(slots.at[j, pl.ds(sid * NPT, NPT)],
                                redb.at[j])
            b2v = b2b[pl.ds(0, 16)]

            def cb(ci, c):
                base = ci * 16
                tot = redb[0, pl.ds(base, 16)]
                for j in range(1, _NSUB):
                    tot = tot + redb[j, pl.ds(base, 16)]
                gslice = g2b[pl.ds(sid * NPT + base, 16)]
                outb[pl.ds(base, 16)] = (
                    disb[pl.ds(base, 16)] * (tot + gslice) + b2v)
                return c

            lax.fori_loop(0, NPT // 16, cb, 0)
            pltpu.sync_copy(outb, out_hbm.at[pl.ds(sid * NPT, NPT)])

    return pl.kernel(
        body,
        out_type=jax.ShapeDtypeStruct((NP,), _F32),
        mesh=_sc_mesh(),
        compiler_params=pltpu.CompilerParams(needs_layout_passes=False),
        scratch_types=[
            pltpu.VMEM((NP,), _F32),
            pltpu.VMEM((EPT,), jnp.int32),
            pltpu.VMEM((EPT,), jnp.int32),
            pltpu.VMEM((NP,), _F32),
            pltpu.VMEM((NPT,), _F32),
            pltpu.VMEM((NPT,), _F32),
            pltpu.VMEM((16,), _F32),
            pltpu.VMEM((_NSUB, NPT), _F32),
            pltpu.VMEM_SHARED((_NSUB, NP), _F32),
        ],
    )


def kernel(x, edge_index, W1, b1, W2, b2):
    N, _ = x.shape
    E = edge_index.shape[1]
    DH = W1.shape[1]
    NP = ((N + 16 * 128 - 1) // (16 * 128)) * (16 * 128)
    CH = 50

    src = edge_index[0]
    dst = edge_index[1]
    src2d = src.reshape(E // CH, CH)
    dst2d = dst.reshape(E // CH, CH)
    x_pad = jnp.pad(x, ((0, NP - N), (0, 0)))

    p = _build_deg_kernel(E, NP)(dst)
    g1, dis2d = _tc_prep(p.T, x_pad, W1)
    accp = _build_prop1_kernel(E, NP, DH, CH)(src2d, dst2d, g1)
    g2_2d = _tc_mid(accp, g1, dis2d, b1, W2)
    g2 = g2_2d[:, 0]
    dis = dis2d[:, 0]
    b2_16 = jnp.broadcast_to(b2, (16,))
    out = _build_prop2_kernel(E, NP)(g2, dis, src, dst, b2_16)
    return out[:N, None]


# speedup vs baseline: 59.7842x; 1.3917x over previous
import functools

import jax
import jax.numpy as jnp
from jax import lax
from jax.experimental import pallas as pl
from jax.experimental.pallas import tpu as pltpu
from jax.experimental.pallas import tpu_sc as plsc

_F32 = jnp.float32
_NT = 32
_NSUB = 16


def _sc_mesh():
    return plsc.VectorSubcoreMesh(core_axis_name="c", subcore_axis_name="s")


@functools.lru_cache(maxsize=None)
def _build_deg_kernel(E, NP):
    EPT = E // _NT

    def body(dst_hbm, out_hbm, dstb, acc):
        cid = lax.axis_index("c")
        sid = lax.axis_index("s")
        t = cid * _NSUB + sid
        zero = jnp.zeros((16,), _F32)

        def zb(i, c):
            acc[pl.ds(i * 16, 16)] = zero
            return c

        lax.fori_loop(0, NP // 16, zb, 0)
        pltpu.sync_copy(dst_hbm.at[pl.ds(t * EPT, EPT)], dstb)
        ones = jnp.ones((16,), _F32)

        def eb(i, c):
            idx = dstb[pl.ds(i * 16, 16)]
            plsc.addupdate_scatter(acc, [idx], ones)
            return c

        lax.fori_loop(0, EPT // 16, eb, 0)
        pltpu.sync_copy(acc, out_hbm.at[t])

    return pl.kernel(
        body,
        out_type=jax.ShapeDtypeStruct((_NT, NP), _F32),
        mesh=_sc_mesh(),
        compiler_params=pltpu.CompilerParams(needs_layout_passes=False),
        scratch_types=[
            pltpu.VMEM((EPT,), jnp.int32),
            pltpu.VMEM((NP,), _F32),
        ],
    )


def _tc_prep(p_t, x_pad, W1):
    NP = x_pad.shape[0]
    DH = W1.shape[1]

    def body(p_ref, x_ref, w1_ref, g1_ref, dis_ref):
        deg = jnp.sum(p_ref[...], axis=1, keepdims=True) + 1.0
        dis = lax.rsqrt(deg)
        h1 = jnp.dot(x_ref[...], w1_ref[...], preferred_element_type=_F32)
        g1_ref[...] = h1 * dis
        dis_ref[...] = dis

    return pl.pallas_call(
        body,
        out_shape=[
            jax.ShapeDtypeStruct((NP, DH), _F32),
            jax.ShapeDtypeStruct((NP, 1), _F32),
        ],
    )(p_t, x_pad, W1)


@functools.lru_cache(maxsize=None)
def _build_prop1_kernel(E, NP, DH, CH):
    NIT = (E // _NT) // CH
    NPT = NP // _NSUB

    NSLOT = 4

    def body(src_hbm, dst_hbm, g1_hbm, out_hbm,
             srcb, dstb, rows, zbuf, sacc, *sems):
        gsem = sems[:NSLOT]
        ssem = sems[NSLOT:]
        cid = lax.axis_index("c")
        sid = lax.axis_index("s")
        t = cid * _NSUB + sid
        pltpu.sync_copy(src_hbm.at[pl.ds(t * NIT, NIT)], srcb)
        pltpu.sync_copy(dst_hbm.at[pl.ds(t * NIT, NIT)], dstb)
        zero = jnp.zeros((16,), _F32)

        def zb(i, c):
            def zc(j, c2):
                zbuf[i, pl.ds(j * 16, 16)] = zero
                return c2
            lax.fori_loop(0, DH // 16, zc, 0)
            return c

        lax.fori_loop(0, 128, zb, 0)
        for r in range(NPT // 128):
            pltpu.sync_copy(zbuf, sacc.at[pl.ds(sid * NPT + r * 128, 128)])
        plsc.subcore_barrier()

        def start_g(i, s):
            pltpu.async_copy(g1_hbm.at[srcb.at[i]], rows.at[s], gsem[s])

        def wait_g(s):
            pltpu.make_async_copy(
                g1_hbm.at[srcb.at[0]], rows.at[s], gsem[s]).wait()

        def start_s(i, s):
            pltpu.async_copy(rows.at[s], sacc.at[dstb.at[i]], ssem[s],
                             add=True)

        def wait_s(s):
            pltpu.make_async_copy(
                rows.at[s], sacc.at[dstb.at[0]], ssem[s]).wait()

        for s in range(NSLOT):
            start_g(s, s)

        def eb(j, c):
            i0 = NSLOT * j
            for s in range(NSLOT):
                wait_g(s)
                start_s(i0 + s, s)

            @pl.when(j < NIT // NSLOT - 1)
            def _():
                for s in range(NSLOT):
                    wait_s(s)
                    start_g(i0 + NSLOT + s, s)

            return c

        lax.fori_loop(0, NIT // NSLOT, eb, 0)
        for s in range(NSLOT):
            wait_s(s)
        plsc.subcore_barrier()
        pltpu.sync_copy(sacc.at[pl.ds(sid * NPT, NPT)],
                        out_hbm.at[cid, pl.ds(sid * NPT, NPT)])

    return pl.kernel(
        body,
        out_type=jax.ShapeDtypeStruct((2, NP, DH), _F32),
        mesh=_sc_mesh(),
        compiler_params=pltpu.CompilerParams(
            needs_layout_passes=False, use_tc_tiling_on_sc=False),
        scratch_types=(
            [pltpu.VMEM((NIT, CH), jnp.int32),
             pltpu.VMEM((NIT, CH), jnp.int32),
             pltpu.VMEM((NSLOT, CH, DH), _F32),
             pltpu.VMEM((128, DH), _F32),
             pltpu.VMEM_SHARED((NP, DH), _F32)]
            + [pltpu.SemaphoreType.DMA] * (2 * NSLOT)
        ),
    )


def _tc_mid(accp, g1, dis2d, b1, W2):
    NP, DH = g1.shape

    def body(a_ref, g1_ref, dis_ref, b1_ref, w2_ref, g2_ref):
        acc = a_ref[0] + a_ref[1] + g1_ref[...]
        dis = dis_ref[...]
        out1 = acc * dis + b1_ref[...]
        h = jnp.maximum(out1, 0.0)
        h2 = jnp.dot(h, w2_ref[...], preferred_element_type=_F32)
        g2_ref[...] = dis * h2

    return pl.pallas_call(
        body,
        out_shape=jax.ShapeDtypeStruct((NP, 1), _F32),
    )(accp, g1, dis2d, b1.reshape(1, DH), W2)


@functools.lru_cache(maxsize=None)
def _build_prop2_kernel(E, NP):
    EPT = E // _NSUB
    NPT = NP // _NSUB

    def body(g2_hbm, dis_hbm, src_hbm, dst_hbm, b2_hbm, out_hbm,
             g2b, srcb, dstb, acc2, disb, outb, b2b, redb, slots):
        cid = lax.axis_index("c")
        sid = lax.axis_index("s")

        @pl.when(cid == 0)
        def _():
            pltpu.sync_copy(g2_hbm, g2b)
            zero = jnp.zeros((16,), _F32)

            def zb(i, c):
                acc2[pl.ds(i * 16, 16)] = zero
                return c

            lax.fori_loop(0, NP // 16, zb, 0)
            pltpu.sync_copy(src_hbm.at[pl.ds(sid * EPT, EPT)], srcb)
            pltpu.sync_copy(dst_hbm.at[pl.ds(sid * EPT, EPT)], dstb)

            def eb(i, c):
                s = srcb[pl.ds(i * 16, 16)]
                d = dstb[pl.ds(i * 16, 16)]
                v = plsc.load_gather(g2b, [s])
                plsc.addupdate_scatter(acc2, [d], v)
                return c

            lax.fori_loop(0, EPT // 16, eb, 0)
            pltpu.sync_copy(acc2, slots.at[sid])
            plsc.subcore_barrier()
            pltpu.sync_copy(dis_hbm.at[pl.ds(sid * NPT, NPT)], disb)
            pltpu.sync_copy(b2_hbm, b2b)
            for j in range(_NSUB):
                pltpu.sync_copy(slots.at[j, pl.ds(sid * NPT, NPT)],
                                redb.at[j])
            b2v = b2b[pl.ds(0, 16)]

            def cb(ci, c):
                base = ci * 16
                tot = redb[0, pl.ds(base, 16)]
                for j in range(1, _NSUB):
                    tot = tot + redb[j, pl.ds(base, 16)]
                gslice = g2b[pl.ds(sid * NPT + base, 16)]
                outb[pl.ds(base, 16)] = (
                    disb[pl.ds(base, 16)] * (tot + gslice) + b2v)
                return c

            lax.fori_loop(0, NPT // 16, cb, 0)
            pltpu.sync_copy(outb, out_hbm.at[pl.ds(sid * NPT, NPT)])

    return pl.kernel(
        body,
        out_type=jax.ShapeDtypeStruct((NP,), _F32),
        mesh=_sc_mesh(),
        compiler_params=pltpu.CompilerParams(needs_layout_passes=False),
        scratch_types=[
            pltpu.VMEM((NP,), _F32),
            pltpu.VMEM((EPT,), jnp.int32),
            pltpu.VMEM((EPT,), jnp.int32),
            pltpu.VMEM((NP,), _F32),
            pltpu.VMEM((NPT,), _F32),
            pltpu.VMEM((NPT,), _F32),
            pltpu.VMEM((16,), _F32),
            pltpu.VMEM((_NSUB, NPT), _F32),
            pltpu.VMEM_SHARED((_NSUB, NP), _F32),
        ],
    )


def kernel(x, edge_index, W1, b1, W2, b2):
    N, _ = x.shape
    E = edge_index.shape[1]
    DH = W1.shape[1]
    NP = ((N + 16 * 128 - 1) // (16 * 128)) * (16 * 128)
    CH = 125

    src = edge_index[0]
    dst = edge_index[1]
    src2d = src.reshape(E // CH, CH)
    dst2d = dst.reshape(E // CH, CH)
    x_pad = jnp.pad(x, ((0, NP - N), (0, 0)))

    p = _build_deg_kernel(E, NP)(dst)
    g1, dis2d = _tc_prep(p.T, x_pad, W1)
    accp = _build_prop1_kernel(E, NP, DH, CH)(src2d, dst2d, g1)
    g2_2d = _tc_mid(accp, g1, dis2d, b1, W2)
    g2 = g2_2d[:, 0]
    dis = dis2d[:, 0]
    b2_16 = jnp.broadcast_to(b2, (16,))
    out = _build_prop2_kernel(E, NP)(g2, dis, src, dst, b2_16)
    return out[:N, None]


# speedup vs baseline: 66.1578x; 1.1066x over previous
import functools

import jax
import jax.numpy as jnp
from jax import lax
from jax.experimental import pallas as pl
from jax.experimental.pallas import tpu as pltpu
from jax.experimental.pallas import tpu_sc as plsc

_F32 = jnp.float32
_NT = 32
_NSUB = 16


def _sc_mesh():
    return plsc.VectorSubcoreMesh(core_axis_name="c", subcore_axis_name="s")


_SC_PARAMS = dict(
    compiler_params=pltpu.CompilerParams(
        needs_layout_passes=False, use_tc_tiling_on_sc=False),
)


@functools.lru_cache(maxsize=None)
def _build_deg_kernel(E, NP):
    EPT = E // _NT

    def body(ei_hbm, out_hbm, dstb, acc):
        cid = lax.axis_index("c")
        sid = lax.axis_index("s")
        t = cid * _NSUB + sid
        zero = jnp.zeros((16,), _F32)

        def zb(i, c):
            acc[pl.ds(i * 16, 16)] = zero
            return c

        lax.fori_loop(0, NP // 16, zb, 0)
        pltpu.sync_copy(ei_hbm.at[1, pl.ds(t * EPT, EPT)], dstb)
        ones = jnp.ones((16,), _F32)

        def eb(i, c):
            base = i * 80
            for u in range(5):
                idx = dstb[pl.ds(base + u * 16, 16)]
                plsc.addupdate_scatter(acc, [idx], ones)
            return c

        lax.fori_loop(0, EPT // 80, eb, 0)
        pltpu.sync_copy(acc, out_hbm.at[t])

    return pl.kernel(
        body,
        out_type=jax.ShapeDtypeStruct((_NT, NP), _F32),
        mesh=_sc_mesh(),
        scratch_types=[
            pltpu.VMEM((EPT,), jnp.int32),
            pltpu.VMEM((NP,), _F32),
        ],
        **_SC_PARAMS,
    )


def _tc_prep(p, x, W1, NP):
    N = x.shape[0]
    DH = W1.shape[1]

    def body(p_ref, x_ref, w1_ref, g1_ref, dis2_ref, dis1_ref):
        deg = jnp.sum(p_ref[...], axis=0) + 1.0
        dis = lax.rsqrt(deg)
        dis1_ref[...] = dis
        dis2_ref[...] = dis[:, None]
        h1 = jnp.dot(x_ref[...], w1_ref[...], preferred_element_type=_F32)
        g1_ref[pl.ds(0, N), :] = h1 * dis[:N][:, None]

    return pl.pallas_call(
        body,
        out_shape=[
            jax.ShapeDtypeStruct((NP, DH), _F32),
            jax.ShapeDtypeStruct((NP, 1), _F32),
            jax.ShapeDtypeStruct((NP,), _F32),
        ],
    )(p, x, W1)


@functools.lru_cache(maxsize=None)
def _build_prop1_kernel(E, NP, DH, CH, NSLOT):
    EPT = E // _NT
    NIT = EPT // CH
    NPT = NP // _NSUB

    def body(ei_hbm, g1_hbm, out_hbm, srcb, dstb, rows, zbuf, sacc, *sems):
        gsem = sems[:NSLOT]
        ssem = sems[NSLOT:]
        cid = lax.axis_index("c")
        sid = lax.axis_index("s")
        t = cid * _NSUB + sid
        pltpu.sync_copy(ei_hbm.at[0, pl.ds(t * EPT, EPT)], srcb)
        pltpu.sync_copy(ei_hbm.at[1, pl.ds(t * EPT, EPT)], dstb)
        zero = jnp.zeros((16,), _F32)

        def zb(i, c):
            def zc(j, c2):
                zbuf[i, pl.ds(j * 16, 16)] = zero
                return c2
            lax.fori_loop(0, DH // 16, zc, 0)
            return c

        lax.fori_loop(0, 128, zb, 0)
        for r in range(NPT // 128):
            pltpu.sync_copy(zbuf, sacc.at[pl.ds(sid * NPT + r * 128, 128)])
        plsc.subcore_barrier()

        def start_g(i, s):
            pltpu.async_copy(g1_hbm.at[srcb.at[pl.ds(i * CH, CH)]],
                             rows.at[s], gsem[s])

        def wait_g(s):
            pltpu.make_async_copy(g1_hbm.at[srcb.at[pl.ds(0, CH)]],
                                  rows.at[s], gsem[s]).wait()

        def start_s(i, s):
            pltpu.async_copy(rows.at[s],
                             sacc.at[dstb.at[pl.ds(i * CH, CH)]],
                             ssem[s], add=True)

        def wait_s(s):
            pltpu.make_async_copy(rows.at[s],
                                  sacc.at[dstb.at[pl.ds(0, CH)]],
                                  ssem[s]).wait()

        for s in range(NSLOT):
            start_g(s, s)

        def eb(j, c):
            i0 = NSLOT * j
            for s in range(NSLOT):
                wait_g(s)
                start_s(i0 + s, s)

            @pl.when(j < NIT // NSLOT - 1)
            def _():
                for s in range(NSLOT):
                    wait_s(s)
                    start_g(i0 + NSLOT + s, s)

            return c

        lax.fori_loop(0, NIT // NSLOT, eb, 0)
        for s in range(NSLOT):
            wait_s(s)
        plsc.subcore_barrier()
        pltpu.sync_copy(sacc.at[pl.ds(sid * NPT, NPT)],
                        out_hbm.at[cid, pl.ds(sid * NPT, NPT)])

    return pl.kernel(
        body,
        out_type=jax.ShapeDtypeStruct((2, NP, DH), _F32),
        mesh=_sc_mesh(),
        scratch_types=(
            [pltpu.VMEM((EPT,), jnp.int32),
             pltpu.VMEM((EPT,), jnp.int32),
             pltpu.VMEM((NSLOT, CH, DH), _F32),
             pltpu.VMEM((128, DH), _F32),
             pltpu.VMEM_SHARED((NP, DH), _F32)]
            + [pltpu.SemaphoreType.DMA] * (2 * NSLOT)
        ),
        **_SC_PARAMS,
    )


def _tc_mid(accp, g1, dis2d, b1, W2, b2):
    NP, DH = g1.shape

    def body(a_ref, g1_ref, dis_ref, b1_ref, w2_ref, b2_ref, g2_ref, c2_ref):
        acc = a_ref[0] + a_ref[1] + g1_ref[...]
        dis = dis_ref[...]
        out1 = acc * dis + b1_ref[...]
        h = jnp.maximum(out1, 0.0)
        h2 = jnp.dot(h, w2_ref[...], preferred_element_type=_F32)
        g2 = dis * h2
        g2_ref[...] = g2[:, 0]
        c2_ref[...] = (dis * g2 + b2_ref[...])[:, 0]

    return pl.pallas_call(
        body,
        out_shape=[
            jax.ShapeDtypeStruct((NP,), _F32),
            jax.ShapeDtypeStruct((NP,), _F32),
        ],
    )(accp, g1, dis2d, b1.reshape(1, DH), W2, b2.reshape(1, 1))


@functools.lru_cache(maxsize=None)
def _build_prop2_kernel(E, NP):
    EPT = E // _NSUB
    NPT = NP // _NSUB

    def body(g2_hbm, dis_hbm, c2_hbm, ei_hbm, out_hbm,
             g2b, srcb, dstb, acc2, disb, c2b, outb, redb, slots):
        cid = lax.axis_index("c")
        sid = lax.axis_index("s")

        @pl.when(cid == 0)
        def _():
            pltpu.sync_copy(g2_hbm, g2b)
            zero = jnp.zeros((16,), _F32)

            def zb(i, c):
                acc2[pl.ds(i * 16, 16)] = zero
                return c

            lax.fori_loop(0, NP // 16, zb, 0)
            pltpu.sync_copy(ei_hbm.at[0, pl.ds(sid * EPT, EPT)], srcb)
            pltpu.sync_copy(ei_hbm.at[1, pl.ds(sid * EPT, EPT)], dstb)

            def eb(i, c):
                base = i * 80
                for u in range(5):
                    s = srcb[pl.ds(base + u * 16, 16)]
                    d = dstb[pl.ds(base + u * 16, 16)]
                    v = plsc.load_gather(g2b, [s])
                    plsc.addupdate_scatter(acc2, [d], v)
                return c

            lax.fori_loop(0, EPT // 80, eb, 0)
            pltpu.sync_copy(acc2, slots.at[sid])
            plsc.subcore_barrier()
            pltpu.sync_copy(dis_hbm.at[pl.ds(sid * NPT, NPT)], disb)
            pltpu.sync_copy(c2_hbm.at[pl.ds(sid * NPT, NPT)], c2b)
            for j in range(_NSUB):
                pltpu.sync_copy(slots.at[j, pl.ds(sid * NPT, NPT)],
                                redb.at[j])

            def cb(ci, c):
                base = ci * 16
                tot = redb[0, pl.ds(base, 16)]
                for j in range(1, _NSUB):
                    tot = tot + redb[j, pl.ds(base, 16)]
                outb[pl.ds(base, 16)] = (
                    disb[pl.ds(base, 16)] * tot + c2b[pl.ds(base, 16)])
                return c

            lax.fori_loop(0, NPT // 16, cb, 0)
            pltpu.sync_copy(outb, out_hbm.at[pl.ds(sid * NPT, NPT)])

    return pl.kernel(
        body,
        out_type=jax.ShapeDtypeStruct((NP,), _F32),
        mesh=_sc_mesh(),
        scratch_types=[
            pltpu.VMEM((NP,), _F32),
            pltpu.VMEM((EPT,), jnp.int32),
            pltpu.VMEM((EPT,), jnp.int32),
            pltpu.VMEM((NP,), _F32),
            pltpu.VMEM((NPT,), _F32),
            pltpu.VMEM((NPT,), _F32),
            pltpu.VMEM((NPT,), _F32),
            pltpu.VMEM((_NSUB, NPT), _F32),
            pltpu.VMEM_SHARED((_NSUB, NP), _F32),
        ],
        **_SC_PARAMS,
    )


def kernel(x, edge_index, W1, b1, W2, b2):
    N, _ = x.shape
    E = edge_index.shape[1]
    DH = W1.shape[1]
    NP = ((N + 16 * 128 - 1) // (16 * 128)) * (16 * 128)
    CH = 80
    NSLOT = 5

    p = _build_deg_kernel(E, NP)(edge_index)
    g1, dis2d, dis = _tc_prep(p, x, W1, NP)
    accp = _build_prop1_kernel(E, NP, DH, CH, NSLOT)(edge_index, g1)
    g2, c2 = _tc_mid(accp, g1, dis2d, b1, W2, b2)
    out = _build_prop2_kernel(E, NP)(g2, dis, c2, edge_index)
    return out[:N, None]


# speedup vs baseline: 70.7536x; 1.0695x over previous
import functools

import jax
import jax.numpy as jnp
from jax import lax
from jax.experimental import pallas as pl
from jax.experimental.pallas import tpu as pltpu
from jax.experimental.pallas import tpu_sc as plsc

_F32 = jnp.float32
_NT = 32
_NSUB = 16


def _sc_mesh():
    return plsc.VectorSubcoreMesh(core_axis_name="c", subcore_axis_name="s")


_SC_PARAMS = dict(
    compiler_params=pltpu.CompilerParams(
        needs_layout_passes=False, use_tc_tiling_on_sc=False),
)


@functools.lru_cache(maxsize=None)
def _build_deg_kernel(E, NP):
    EPT = E // _NT

    def body(ei_hbm, out_hbm, dstb, acc):
        cid = lax.axis_index("c")
        sid = lax.axis_index("s")
        t = cid * _NSUB + sid
        zero = jnp.zeros((16,), _F32)

        def zb(i, c):
            acc[pl.ds(i * 16, 16)] = zero
            return c

        lax.fori_loop(0, NP // 16, zb, 0)
        pltpu.sync_copy(ei_hbm.at[1, pl.ds(t * EPT, EPT)], dstb)
        ones = jnp.ones((16,), _F32)

        def eb(i, c):
            base = i * 80
            for u in range(5):
                idx = dstb[pl.ds(base + u * 16, 16)]
                plsc.addupdate_scatter(acc, [idx], ones)
            return c

        lax.fori_loop(0, EPT // 80, eb, 0)
        pltpu.sync_copy(acc, out_hbm.at[t])

    return pl.kernel(
        body,
        out_type=jax.ShapeDtypeStruct((_NT, NP), _F32),
        mesh=_sc_mesh(),
        scratch_types=[
            pltpu.VMEM((EPT,), jnp.int32),
            pltpu.VMEM((NP,), _F32),
        ],
        **_SC_PARAMS,
    )


def _tc_prep(p, x, W1, NP):
    N = x.shape[0]
    DH = W1.shape[1]

    def body(p_ref, x_ref, w1_ref, g1_ref, dis2_ref, dis1_ref):
        deg = jnp.sum(p_ref[...], axis=0) + 1.0
        dis = lax.rsqrt(deg)
        dis1_ref[...] = dis
        dis2_ref[...] = dis[:, None]
        h1 = jnp.dot(x_ref[...], w1_ref[...], preferred_element_type=_F32)
        g1_ref[pl.ds(0, N), :] = h1 * dis[:N][:, None]

    return pl.pallas_call(
        body,
        out_shape=[
            jax.ShapeDtypeStruct((NP, DH), _F32),
            jax.ShapeDtypeStruct((NP, 1), _F32),
            jax.ShapeDtypeStruct((NP,), _F32),
        ],
    )(p, x, W1)


@functools.lru_cache(maxsize=None)
def _build_prop1_kernel(E, NP, DH, CH, NSLOT):
    EPT = E // _NT
    NIT = EPT // CH
    NPT = NP // _NSUB

    def body(ei_hbm, g1_hbm, out_hbm, srcb, dstb, rows, zbuf, sacc, *sems):
        gsem = sems[:NSLOT]
        ssem = sems[NSLOT:]
        cid = lax.axis_index("c")
        sid = lax.axis_index("s")
        t = cid * _NSUB + sid
        pltpu.sync_copy(ei_hbm.at[0, pl.ds(t * EPT, EPT)], srcb)
        pltpu.sync_copy(ei_hbm.at[1, pl.ds(t * EPT, EPT)], dstb)
        zero = jnp.zeros((16,), _F32)

        def zb(i, c):
            def zc(j, c2):
                zbuf[i, pl.ds(j * 16, 16)] = zero
                return c2
            lax.fori_loop(0, DH // 16, zc, 0)
            return c

        lax.fori_loop(0, 128, zb, 0)
        for r in range(NPT // 128):
            pltpu.sync_copy(zbuf, sacc.at[pl.ds(sid * NPT + r * 128, 128)])
        plsc.subcore_barrier()

        def start_g(i, s):
            pltpu.async_copy(g1_hbm.at[srcb.at[pl.ds(i * CH, CH)]],
                             rows.at[s], gsem[s])

        def wait_g(s):
            pltpu.make_async_copy(g1_hbm.at[srcb.at[pl.ds(0, CH)]],
                                  rows.at[s], gsem[s]).wait()

        def start_s(i, s):
            pltpu.async_copy(rows.at[s],
                             sacc.at[dstb.at[pl.ds(i * CH, CH)]],
                             ssem[s], add=True)

        def wait_s(s):
            pltpu.make_async_copy(rows.at[s],
                                  sacc.at[dstb.at[pl.ds(0, CH)]],
                                  ssem[s]).wait()

        for s in range(NSLOT):
            start_g(s, s)

        def eb(j, c):
            i0 = NSLOT * j
            for s in range(NSLOT):
                wait_g(s)
                start_s(i0 + s, s)

            @pl.when(j < NIT // NSLOT - 1)
            def _():
                for s in range(NSLOT):
                    wait_s(s)
                    start_g(i0 + NSLOT + s, s)

            return c

        lax.fori_loop(0, NIT // NSLOT, eb, 0)
        for s in range(NSLOT):
            wait_s(s)
        plsc.subcore_barrier()
        pltpu.sync_copy(sacc.at[pl.ds(sid * NPT, NPT)],
                        out_hbm.at[cid, pl.ds(sid * NPT, NPT)])

    return pl.kernel(
        body,
        out_type=jax.ShapeDtypeStruct((2, NP, DH), _F32),
        mesh=_sc_mesh(),
        scratch_types=(
            [pltpu.VMEM((EPT,), jnp.int32),
             pltpu.VMEM((EPT,), jnp.int32),
             pltpu.VMEM((NSLOT, CH, DH), _F32),
             pltpu.VMEM((128, DH), _F32),
             pltpu.VMEM_SHARED((NP, DH), _F32)]
            + [pltpu.SemaphoreType.DMA] * (2 * NSLOT)
        ),
        **_SC_PARAMS,
    )


def _tc_mid(accp, g1, dis2d, dis1, b1, W2, b2):
    NP, DH = g1.shape

    def body(a_ref, g1_ref, dis2_ref, dis1_ref, b1_ref, w2_ref, b2_ref,
             g2_ref, c2_ref):
        acc = a_ref[0] + a_ref[1] + g1_ref[...]
        out1 = acc * dis2_ref[...] + b1_ref[...]
        h = jnp.maximum(out1, 0.0)
        h2 = jnp.sum(h * w2_ref[...], axis=1)
        dis = dis1_ref[...]
        g2 = dis * h2
        g2_ref[...] = g2
        c2_ref[...] = dis * g2 + b2_ref[...]

    return pl.pallas_call(
        body,
        out_shape=[
            jax.ShapeDtypeStruct((NP,), _F32),
            jax.ShapeDtypeStruct((NP,), _F32),
        ],
    )(accp, g1, dis2d, dis1, b1.reshape(1, DH), W2.reshape(1, DH),
      b2.reshape(1))


@functools.lru_cache(maxsize=None)
def _build_prop2_kernel(E, NP):
    EPT = E // _NT
    NPT = NP // _NSUB

    def body(g2_hbm, ei_hbm, out_hbm, g2b, srcb, dstb, acc2, outb, redb,
             slots):
        cid = lax.axis_index("c")
        sid = lax.axis_index("s")
        t = cid * _NSUB + sid
        pltpu.sync_copy(g2_hbm, g2b)
        zero = jnp.zeros((16,), _F32)

        def zb(i, c):
            base = i * 64
            for u in range(4):
                acc2[pl.ds(base + u * 16, 16)] = zero
            return c

        lax.fori_loop(0, NP // 64, zb, 0)
        pltpu.sync_copy(ei_hbm.at[0, pl.ds(t * EPT, EPT)], srcb)
        pltpu.sync_copy(ei_hbm.at[1, pl.ds(t * EPT, EPT)], dstb)

        def eb(i, c):
            base = i * 80
            for u in range(5):
                s = srcb[pl.ds(base + u * 16, 16)]
                d = dstb[pl.ds(base + u * 16, 16)]
                v = plsc.load_gather(g2b, [s])
                plsc.addupdate_scatter(acc2, [d], v)
            return c

        lax.fori_loop(0, EPT // 80, eb, 0)
        pltpu.sync_copy(acc2, slots.at[sid])
        plsc.subcore_barrier()
        for j in range(_NSUB):
            pltpu.sync_copy(slots.at[j, pl.ds(sid * NPT, NPT)], redb.at[j])

        def cb(ci, c):
            base = ci * 16
            tot = redb[0, pl.ds(base, 16)]
            for j in range(1, _NSUB):
                tot = tot + redb[j, pl.ds(base, 16)]
            outb[pl.ds(base, 16)] = tot
            return c

        lax.fori_loop(0, NPT // 16, cb, 0)
        pltpu.sync_copy(outb, out_hbm.at[cid, pl.ds(sid * NPT, NPT)])

    return pl.kernel(
        body,
        out_type=jax.ShapeDtypeStruct((2, NP), _F32),
        mesh=_sc_mesh(),
        scratch_types=[
            pltpu.VMEM((NP,), _F32),
            pltpu.VMEM((EPT,), jnp.int32),
            pltpu.VMEM((EPT,), jnp.int32),
            pltpu.VMEM((NP,), _F32),
            pltpu.VMEM((NPT,), _F32),
            pltpu.VMEM((_NSUB, NPT), _F32),
            pltpu.VMEM_SHARED((_NSUB, NP), _F32),
        ],
        **_SC_PARAMS,
    )


def _tc_final(p2, dis1, c2):
    NP = dis1.shape[0]

    def body(p_ref, dis_ref, c2_ref, out_ref):
        out_ref[...] = (dis_ref[...] * (p_ref[0] + p_ref[1])
                        + c2_ref[...])

    return pl.pallas_call(
        body,
        out_shape=jax.ShapeDtypeStruct((NP,), _F32),
    )(p2, dis1, c2)


def kernel(x, edge_index, W1, b1, W2, b2):
    N, _ = x.shape
    E = edge_index.shape[1]
    DH = W1.shape[1]
    NP = ((N + 16 * 128 - 1) // (16 * 128)) * (16 * 128)
    CH = 80
    NSLOT = 5

    p = _build_deg_kernel(E, NP)(edge_index)
    g1, dis2d, dis = _tc_prep(p, x, W1, NP)
    accp = _build_prop1_kernel(E, NP, DH, CH, NSLOT)(edge_index, g1)
    g2, c2 = _tc_mid(accp, g1, dis2d, dis, b1, W2, b2)
    p2 = _build_prop2_kernel(E, NP)(g2, edge_index)
    out = _tc_final(p2, dis, c2)
    return out[:N, None]
